# baseline (speedup 1.0000x reference)
import functools

import jax
import jax.numpy as jnp
from jax import lax
from jax.experimental import pallas as pl
from jax.experimental.pallas import tpu as pltpu
from jax.experimental.pallas import tpu_sc as plsc

NJ = 17
SUBSETS = [[0, 5, 6, 11, 12], [7, 8, 13, 14], [9, 10, 15, 16]]
_ms = [frozenset(s) for s in SUBSETS]
NB = [sorted(_ms[0] | _ms[1]), sorted(_ms[0] | _ms[1] | _ms[2]), sorted(_ms[1] | _ms[2])]
MASKS = [sorted(s) for s in _ms]
NBL = [len(x) for x in NB]
NML = [len(x) for x in MASKS]
NW = 32


def _tc1_body(kxyt_ref, sct_ref, w40_ref, w41_ref, w42_ref, we_ref, be_ref,
              c_ref, a_ref, s_ref):
    kxyt = kxyt_ref[...]
    sct = sct_ref[...]
    bn = kxyt.shape[1]
    row = jax.lax.broadcasted_iota(jnp.int32, (2 * NJ, bn), 0)
    is_x = (row % 2) == 0
    big = jnp.float32(1e30)
    mnx = jnp.min(jnp.where(is_x, kxyt, big), axis=0, keepdims=True)
    mny = jnp.min(jnp.where(is_x, big, kxyt), axis=0, keepdims=True)
    mxx = jnp.max(jnp.where(is_x, kxyt, -big), axis=0, keepdims=True)
    mxy = jnp.max(jnp.where(is_x, -big, kxyt), axis=0, keepdims=True)
    mn = jnp.where(is_x, mnx, mny)
    inv = jnp.where(is_x, 1.0 / (mxx - mnx + 1e-6), 1.0 / (mxy - mny + 1e-6))
    nt = (kxyt - mn) * inv

    m = NJ * bn
    xrow = jnp.concatenate([nt[2 * j:2 * j + 1, :] for j in range(NJ)], axis=1)
    yrow = jnp.concatenate([nt[2 * j + 1:2 * j + 2, :] for j in range(NJ)], axis=1)
    srow = jnp.concatenate([sct[j:j + 1, :] for j in range(NJ)], axis=1)
    onerow = jnp.ones((1, m), dtype=jnp.float32)
    p4 = jnp.concatenate([xrow, yrow, srow, onerow], axis=0)

    we = we_ref[...]
    we2 = we[:, 64:128]
    ab = jnp.concatenate([we[:, 0:64] - we2, be_ref[...]], axis=1)
    w4s = [w40_ref, w41_ref, w42_ref]

    for lvl in range(3):
        h = jnp.dot(w4s[lvl][...], p4, preferred_element_type=jnp.float32)
        fm = jnp.maximum(h, 0.0) * srow
        csl = lambda arr, j: arr[:, j * bn:(j + 1) * bn]
        ssum = functools.reduce(jnp.add, [csl(fm, j) for j in MASKS[lvl]])
        f5 = jnp.concatenate([fm, onerow], axis=0)
        ct = jnp.dot(we2, fm, preferred_element_type=jnp.float32)
        at = jnp.dot(ab, f5, preferred_element_type=jnp.float32)
        c_ref[lvl, :, 0:NBL[lvl] * 64] = jnp.concatenate(
            [jnp.transpose(csl(ct, k), (1, 0)) for k in NB[lvl]], axis=1)
        a_ref[lvl, :, 0:NML[lvl] * 64] = jnp.concatenate(
            [jnp.transpose(csl(at, j), (1, 0)) for j in MASKS[lvl]], axis=1)
        s_ref[64 * lvl:64 * lvl + 64, :] = ssum


def _tc1(kxyt, sct, w4, We, be_col):
    n = kxyt.shape[1]
    rep = lambda shape: pl.BlockSpec(shape, lambda: tuple(0 for _ in shape))
    return pl.pallas_call(
        _tc1_body,
        in_specs=[rep((2 * NJ, n)), rep((NJ, n)),
                  rep((64, 4)), rep((64, 4)), rep((64, 4)),
                  rep((64, 128)), rep((64, 1))],
        out_specs=[rep((3, n, NJ * 64)), rep((3, n, 5 * 64)),
                   rep((192, n))],
        out_shape=[jax.ShapeDtypeStruct((3, n, NJ * 64), jnp.float32),
                   jax.ShapeDtypeStruct((3, n, 5 * 64), jnp.float32),
                   jax.ShapeDtypeStruct((192, n), jnp.float32)],
    )(kxyt, sct, w4[0], w4[1], w4[2], We, be_col)


def _sc_agg(n):
    spw = n // NW
    mesh = plsc.VectorSubcoreMesh(core_axis_name="c", subcore_axis_name="s")

    @functools.partial(
        pl.kernel, mesh=mesh,
        out_type=jax.ShapeDtypeStruct((n, 192), jnp.float32),
        scratch_types=[
            pltpu.VMEM((spw, NJ * 64), jnp.float32),
            pltpu.VMEM((spw, 5 * 64), jnp.float32),
            pltpu.VMEM((spw, 192), jnp.float32),
        ],
    )
    def agg(c_hbm, a_hbm, z_hbm, cv, av, zv):
        wid = lax.axis_index("s") * 2 + lax.axis_index("c")
        s0 = wid * spw
        for lvl in range(3):
            nbl, nml = NBL[lvl], NML[lvl]
            pltpu.sync_copy(c_hbm.at[lvl, pl.ds(s0, spw), :], cv)
            pltpu.sync_copy(a_hbm.at[lvl, pl.ds(s0, spw), :], av)
            apos = [NB[lvl].index(j) for j in MASKS[lvl]]

            def samp_body(s, carry):
                for q in range(4):
                    qs = q * 16
                    cs = [cv[s, pl.ds(i * 64 + qs, 16)] for i in range(nbl)]
                    m1 = jnp.maximum(cs[0], cs[1])
                    m2 = jnp.minimum(cs[0], cs[1])
                    for t in range(2, nbl):
                        m2 = jnp.maximum(m2, jnp.minimum(m1, cs[t]))
                        m1 = jnp.maximum(m1, cs[t])
                    z = None
                    for i in range(nml):
                        cj = cs[apos[i]]
                        zj = jnp.maximum(
                            av[s, pl.ds(i * 64 + qs, 16)]
                            + jnp.where(cj == m1, m2, m1), 0.0)
                        z = zj if z is None else z + zj
                    zv[s, pl.ds(lvl * 64 + qs, 16)] = z
                return carry

            lax.fori_loop(0, spw, samp_body, 0)
        pltpu.sync_copy(zv, z_hbm.at[pl.ds(s0, spw), :])

    return agg


def _tc2_body(s_ref, z_ref, wp_ref, bp_ref, out_ref):
    n = s_ref.shape[1]
    wp = wp_ref[...]
    zt = z_ref[...]
    nt_dims = (((1,), (1,)), ((), ()))
    acc = jnp.broadcast_to(bp_ref[...], (128, n))
    for lvl in range(3):
        ssum = s_ref[64 * lvl:64 * lvl + 64, :]
        zl = zt[:, 64 * lvl:64 * lvl + 64] * (1.0 / float(NML[lvl]))
        acc = acc + jnp.dot(wp[:, 128 * lvl:128 * lvl + 64], ssum,
                            preferred_element_type=jnp.float32)
        acc = acc + jax.lax.dot_general(
            wp[:, 128 * lvl + 64:128 * lvl + 128], zl, nt_dims,
            preferred_element_type=jnp.float32)
    out_ref[...] = jnp.transpose(acc, (1, 0))


def _tc2(sbuf, zbuf, Wp, bp_col):
    n = sbuf.shape[1]
    rep = lambda shape: pl.BlockSpec(shape, lambda: tuple(0 for _ in shape))
    return pl.pallas_call(
        _tc2_body,
        in_specs=[rep((192, n)), rep((n, 192)),
                  rep((128, 384)), rep((128, 1))],
        out_specs=rep((n, 128)),
        out_shape=jax.ShapeDtypeStruct((n, 128), jnp.float32),
    )(sbuf, zbuf, Wp, bp_col)


def kernel(keypoints, scores, W0, b0, W1, b1, W2, b2, We, be, Wp, bp):
    n = keypoints.shape[0]
    kxyt = keypoints.reshape(n, 2 * NJ).T
    sct = scores.T
    w4 = [jnp.concatenate([w, b[:, None]], axis=1)
          for w, b in ((W0, b0), (W1, b1), (W2, b2))]
    cbuf, abuf, sbuf = _tc1(kxyt, sct, w4, We, be[:, None])
    zbuf = _sc_agg(n)(cbuf, abuf)
    return _tc2(sbuf, zbuf, Wp, bp[:, None])


# speedup vs baseline: 4.4968x; 4.4968x over previous
import functools

import jax
import jax.numpy as jnp
from jax.experimental import pallas as pl

NJ = 17
SUBSETS = [[0, 5, 6, 11, 12], [7, 8, 13, 14], [9, 10, 15, 16]]
_ms = [frozenset(s) for s in SUBSETS]
NB = [sorted(_ms[0] | _ms[1]), sorted(_ms[0] | _ms[1] | _ms[2]), sorted(_ms[1] | _ms[2])]
MASKS = [sorted(s) for s in _ms]


def _body(kxyt_ref, sct_ref, w40_ref, w41_ref, w42_ref, we_ref, be_ref,
          wp_ref, bp_ref, out_ref):
    kxyt = kxyt_ref[...]
    sct = sct_ref[...]
    bn = kxyt.shape[1]
    row = jax.lax.broadcasted_iota(jnp.int32, (2 * NJ, bn), 0)
    is_x = (row % 2) == 0
    big = jnp.float32(1e30)
    mnx = jnp.min(jnp.where(is_x, kxyt, big), axis=0, keepdims=True)
    mny = jnp.min(jnp.where(is_x, big, kxyt), axis=0, keepdims=True)
    mxx = jnp.max(jnp.where(is_x, kxyt, -big), axis=0, keepdims=True)
    mxy = jnp.max(jnp.where(is_x, -big, kxyt), axis=0, keepdims=True)
    mn = jnp.where(is_x, mnx, mny)
    inv = jnp.where(is_x, 1.0 / (mxx - mnx + 1e-6), 1.0 / (mxy - mny + 1e-6))
    nt = (kxyt - mn) * inv

    m = NJ * bn
    xrow = jnp.concatenate([nt[2 * j:2 * j + 1, :] for j in range(NJ)], axis=1)
    yrow = jnp.concatenate([nt[2 * j + 1:2 * j + 2, :] for j in range(NJ)], axis=1)
    srow = jnp.concatenate([sct[j:j + 1, :] for j in range(NJ)], axis=1)
    onerow = jnp.ones((1, m), dtype=jnp.float32)
    p4 = jnp.concatenate([xrow, yrow, srow, onerow], axis=0)

    we = we_ref[...]
    we2 = we[:, 64:128]
    ab = jnp.concatenate([we[:, 0:64] - we2, be_ref[...]], axis=1)
    wp = wp_ref[...]
    w4s = [w40_ref, w41_ref, w42_ref]

    acc = jnp.broadcast_to(bp_ref[...], (128, bn))
    for lvl in range(3):
        h = jnp.dot(w4s[lvl][...], p4, preferred_element_type=jnp.float32)
        fm = jnp.maximum(h, 0.0) * srow
        csl = lambda arr, j: arr[:, j * bn:(j + 1) * bn]
        ssum = functools.reduce(jnp.add, [csl(fm, j) for j in MASKS[lvl]])
        f5 = jnp.concatenate([fm, onerow], axis=0)
        ct = jnp.dot(we2, fm, preferred_element_type=jnp.float32)
        at = jnp.dot(ab, f5, preferred_element_type=jnp.float32)
        ks = NB[lvl]
        c = {k: csl(ct, k) for k in ks}
        m1 = jnp.maximum(c[ks[0]], c[ks[1]])
        m2 = jnp.minimum(c[ks[0]], c[ks[1]])
        for k in ks[2:]:
            m2 = jnp.maximum(m2, jnp.minimum(m1, c[k]))
            m1 = jnp.maximum(m1, c[k])
        zsum = functools.reduce(jnp.add, [
            jnp.maximum(csl(at, j) + jnp.where(c[j] == m1, m2, m1), 0.0)
            for j in MASKS[lvl]])
        inv_cnt = 1.0 / float(len(MASKS[lvl]))
        acc = acc + jnp.dot(wp[:, 128 * lvl:128 * lvl + 64], ssum,
                            preferred_element_type=jnp.float32)
        acc = acc + jnp.dot(wp[:, 128 * lvl + 64:128 * lvl + 128],
                            zsum * inv_cnt,
                            preferred_element_type=jnp.float32)
    out_ref[...] = jnp.transpose(acc, (1, 0))


def kernel(keypoints, scores, W0, b0, W1, b1, W2, b2, We, be, Wp, bp):
    n = keypoints.shape[0]
    block_n = 512
    kxyt = keypoints.reshape(n, 2 * NJ).T
    sct = scores.T
    w4 = [jnp.concatenate([w, b[:, None]], axis=1)
          for w, b in ((W0, b0), (W1, b1), (W2, b2))]
    tspec = lambda r: pl.BlockSpec((r, block_n), lambda i: (0, i))

    def rep(shape):
        return pl.BlockSpec(shape, lambda i: tuple(0 for _ in shape))

    return pl.pallas_call(
        _body,
        grid=(n // block_n,),
        in_specs=[
            tspec(2 * NJ), tspec(NJ),
            rep((64, 4)), rep((64, 4)), rep((64, 4)),
            rep((64, 128)), rep((64, 1)),
            rep((128, 384)), rep((128, 1)),
        ],
        out_specs=pl.BlockSpec((block_n, 128), lambda i: (i, 0)),
        out_shape=jax.ShapeDtypeStruct((n, 128), jnp.float32),
    )(kxyt, sct, w4[0], w4[1], w4[2], We, be[:, None], Wp, bp[:, None])
